# baseline (speedup 1.0000x reference)
import functools

import jax
import jax.numpy as jnp
from jax import lax
from jax.experimental import pallas as pl
from jax.experimental.pallas import tpu as pltpu
from jax.experimental.pallas import tpu_sc as plsc

LANES = 16
NUM_CORES = 2
NUM_SUBCORES = 16
CHUNK = 128


def _sc_spmm(num_op, n, d, k_chunks):
    dh = d // NUM_CORES
    qs = dh // LANES
    zcopy = CHUNK
    n_zcopy = -(-n // (NUM_SUBCORES * zcopy))
    rows_per_tile = n_zcopy * zcopy
    n_acc = NUM_SUBCORES * rows_per_tile

    mesh = plsc.VectorSubcoreMesh(core_axis_name="c", subcore_axis_name="s")

    @functools.partial(
        pl.kernel,
        mesh=mesh,
        compiler_params=pltpu.CompilerParams(
            needs_layout_passes=False, use_tc_tiling_on_sc=False),
        out_type=jax.ShapeDtypeStruct((NUM_CORES, n_acc, dh), jnp.float32),
        scratch_types=[
            pltpu.VMEM((num_op, LANES), jnp.float32),
            pltpu.VMEM((k_chunks, CHUNK), jnp.int32),
            pltpu.VMEM((k_chunks, CHUNK), jnp.int32),
            pltpu.VMEM((k_chunks, CHUNK), jnp.float32),
            pltpu.VMEM((CHUNK, dh), jnp.float32),
            pltpu.VMEM((CHUNK, dh), jnp.float32),
            pltpu.VMEM_SHARED((n_acc, dh), jnp.float32),
            pltpu.SemaphoreType.DMA,
            pltpu.SemaphoreType.DMA,
        ],
    )
    def k(x_hbm, src_hbm, dst_hbm, val_hbm, ws_hbm, out_hbm,
          ws_v, src_v, dst_v, val_v, rows_0, rows_1, acc, sem_0, sem_1):
        bufs = (rows_0, rows_1)
        sems = (sem_0, sem_1)
        c = lax.axis_index("c")
        s = lax.axis_index("s")

        def _zrow(r, carry):
            for q in range(qs):
                rows_0[r, pl.ds(q * LANES, LANES)] = jnp.zeros(
                    (LANES,), jnp.float32)
            return carry
        lax.fori_loop(0, CHUNK, _zrow, 0)

        base = s * rows_per_tile
        for z in range(n_zcopy):
            pltpu.sync_copy(rows_0.at[pl.ds(0, zcopy)],
                            acc.at[pl.ds(base + z * zcopy, zcopy)])
        plsc.subcore_barrier()

        pltpu.sync_copy(ws_hbm, ws_v)

        for i in range(num_op):
            pltpu.sync_copy(src_hbm.at[c, i, s], src_v)
            pltpu.sync_copy(dst_hbm.at[i, s], dst_v)
            pltpu.sync_copy(val_hbm.at[i, s], val_v)
            wvec = ws_v[i]

            def _vscale(jj, carry):
                for q in range(CHUNK // LANES):
                    sl = pl.ds(q * LANES, LANES)
                    val_v[jj, sl] = val_v[jj, sl] * wvec
                return carry
            lax.fori_loop(0, k_chunks, _vscale, 0)

            pltpu.async_copy(x_hbm.at[src_v.at[0]], bufs[0], sems[0])

            def _pair(t, carry):
                for b in range(2):
                    j = 2 * t + b
                    rows_b = bufs[b]
                    pltpu.make_async_copy(
                        x_hbm.at[src_v.at[j]], rows_b, sems[b]).wait()

                    @pl.when(j + 1 < k_chunks)
                    def _():
                        pltpu.async_copy(
                            x_hbm.at[src_v.at[j + 1]], bufs[1 - b],
                            sems[1 - b])

                    def _grp(g, cc):
                        vv = val_v[j, pl.ds(g * LANES, LANES)]
                        rbase = g * LANES
                        for l in range(LANES):
                            sval = vv[l]
                            for q in range(qs):
                                sl = pl.ds(q * LANES, LANES)
                                rows_b[rbase + l, sl] = (
                                    rows_b[rbase + l, sl] * sval)
                        return cc
                    lax.fori_loop(0, CHUNK // LANES, _grp, 0)

                    pltpu.sync_copy(rows_b, acc.at[dst_v.at[j]], add=True)
                return carry
            lax.fori_loop(0, k_chunks // 2, _pair, 0)

        plsc.subcore_barrier()
        for z in range(n_zcopy):
            sl = pl.ds(base + z * zcopy, zcopy)
            pltpu.sync_copy(acc.at[sl], out_hbm.at[c].at[sl])

    return k


def kernel(x, adj_indices, adj_values, ws):
    n, d = x.shape
    num_op, _, e = adj_indices.shape
    dh = d // NUM_CORES
    k_chunks = -(-e // (NUM_SUBCORES * CHUNK))
    k_chunks += k_chunks % 2
    e_pad = NUM_SUBCORES * k_chunks * CHUNK
    pad = e_pad - e

    xh = x.reshape(n, NUM_CORES, dh).transpose(1, 0, 2).reshape(
        NUM_CORES * n, dh)
    src = jnp.pad(adj_indices[:, 1, :], ((0, 0), (0, pad))).reshape(
        num_op, NUM_SUBCORES, k_chunks, CHUNK)
    src = jnp.stack([src + cc * n for cc in range(NUM_CORES)])
    dst = jnp.pad(adj_indices[:, 0, :], ((0, 0), (0, pad))).reshape(
        num_op, NUM_SUBCORES, k_chunks, CHUNK)
    val = jnp.pad(adj_values, ((0, 0), (0, pad))).reshape(
        num_op, NUM_SUBCORES, k_chunks, CHUNK)
    wsp = jnp.tile((ws / jnp.float32(num_op))[:, None], (1, LANES))

    parts = _sc_spmm(num_op, n, d, k_chunks)(xh, src, dst, val, wsp)
    return parts.transpose(1, 0, 2).reshape(-1, d)[:n]


# speedup vs baseline: 1.3192x; 1.3192x over previous
import functools

import jax
import jax.numpy as jnp
from jax import lax
from jax.experimental import pallas as pl
from jax.experimental.pallas import tpu as pltpu
from jax.experimental.pallas import tpu_sc as plsc

LANES = 16
NUM_CORES = 2
NUM_SUBCORES = 16
NW = NUM_CORES * NUM_SUBCORES
CHUNK = 64


def _sc_spmm(num_op, n, d, k_chunks):
    qs = d // LANES
    zcopy = CHUNK
    n_zcopy = -(-n // (NUM_SUBCORES * zcopy))
    rows_per_tile = n_zcopy * zcopy
    n_acc = NUM_SUBCORES * rows_per_tile

    mesh = plsc.VectorSubcoreMesh(core_axis_name="c", subcore_axis_name="s")

    @functools.partial(
        pl.kernel,
        mesh=mesh,
        compiler_params=pltpu.CompilerParams(
            needs_layout_passes=False, use_tc_tiling_on_sc=False),
        out_type=jax.ShapeDtypeStruct((NUM_CORES, n_acc, d), jnp.float32),
        scratch_types=[
            pltpu.VMEM((num_op, LANES), jnp.float32),
            pltpu.VMEM((k_chunks, CHUNK), jnp.int32),
            pltpu.VMEM((k_chunks, CHUNK), jnp.int32),
            pltpu.VMEM((k_chunks, CHUNK), jnp.float32),
            pltpu.VMEM((CHUNK, d), jnp.float32),
            pltpu.VMEM((CHUNK, d), jnp.float32),
            pltpu.VMEM_SHARED((n_acc, d), jnp.float32),
            pltpu.SemaphoreType.DMA,
            pltpu.SemaphoreType.DMA,
            pltpu.SemaphoreType.DMA,
            pltpu.SemaphoreType.DMA,
        ],
    )
    def k(x_hbm, src_hbm, dst_hbm, val_hbm, ws_hbm, out_hbm,
          ws_v, src_v, dst_v, val_v, rows_0, rows_1, acc,
          sem_0, sem_1, ssem_0, ssem_1):
        bufs = (rows_0, rows_1)
        sems = (sem_0, sem_1)
        ssems = (ssem_0, ssem_1)
        c = lax.axis_index("c")
        s = lax.axis_index("s")
        wid = s * NUM_CORES + c

        def _zrow(r, carry):
            for q in range(qs):
                rows_0[r, pl.ds(q * LANES, LANES)] = jnp.zeros(
                    (LANES,), jnp.float32)
            return carry
        lax.fori_loop(0, CHUNK, _zrow, 0)

        base = s * rows_per_tile
        for z in range(n_zcopy):
            pltpu.sync_copy(rows_0.at[pl.ds(0, zcopy)],
                            acc.at[pl.ds(base + z * zcopy, zcopy)])
        plsc.subcore_barrier()

        pltpu.sync_copy(ws_hbm, ws_v)

        for i in range(num_op):
            pltpu.sync_copy(src_hbm.at[i, wid], src_v)
            pltpu.sync_copy(dst_hbm.at[i, wid], dst_v)
            pltpu.sync_copy(val_hbm.at[i, wid], val_v)
            wvec = ws_v[i]

            def _vscale(jj, carry):
                for q in range(CHUNK // LANES):
                    sl = pl.ds(q * LANES, LANES)
                    val_v[jj, sl] = val_v[jj, sl] * wvec
                return carry
            lax.fori_loop(0, k_chunks, _vscale, 0)

            pltpu.async_copy(x_hbm.at[src_v.at[0]], bufs[0], sems[0])

            def _pair(t, carry):
                for b in range(2):
                    j = 2 * t + b
                    rows_b = bufs[b]
                    pltpu.make_async_copy(
                        x_hbm.at[src_v.at[j]], rows_b, sems[b]).wait()

                    @pl.when(j >= 1)
                    def _():
                        pltpu.make_async_copy(
                            bufs[1 - b], acc.at[dst_v.at[j - 1]],
                            ssems[1 - b]).wait()

                    @pl.when(j + 1 < k_chunks)
                    def _():
                        pltpu.async_copy(
                            x_hbm.at[src_v.at[j + 1]], bufs[1 - b],
                            sems[1 - b])

                    @plsc.parallel_loop(0, CHUNK // LANES)
                    def _grp(g):
                        vv = val_v[j, pl.ds(g * LANES, LANES)]
                        rbase = g * LANES
                        for l in range(LANES):
                            sval = vv[l]
                            r = rbase + l
                            loaded = [rows_b[r, pl.ds(q * LANES, LANES)]
                                      for q in range(qs)]
                            prods = [v * sval for v in loaded]
                            for q in range(qs):
                                rows_b[r, pl.ds(q * LANES, LANES)] = prods[q]

                    pltpu.async_copy(
                        rows_b, acc.at[dst_v.at[j]], ssems[b], add=True)
                return carry
            lax.fori_loop(0, k_chunks // 2, _pair, 0)

            pltpu.make_async_copy(
                bufs[1], acc.at[dst_v.at[k_chunks - 1]], ssems[1]).wait()

        plsc.subcore_barrier()
        for z in range(n_zcopy):
            sl = pl.ds(base + z * zcopy, zcopy)
            pltpu.sync_copy(acc.at[sl], out_hbm.at[c].at[sl])

    return k


def _combine(p_ref, o_ref):
    o_ref[...] = p_ref[0] + p_ref[1]


def kernel(x, adj_indices, adj_values, ws):
    n, d = x.shape
    num_op, _, e = adj_indices.shape
    k_chunks = -(-e // (NW * CHUNK))
    k_chunks += k_chunks % 2
    e_pad = NW * k_chunks * CHUNK
    pad = e_pad - e

    src = jnp.pad(adj_indices[:, 1, :], ((0, 0), (0, pad))).reshape(
        num_op, NW, k_chunks, CHUNK)
    dst = jnp.pad(adj_indices[:, 0, :], ((0, 0), (0, pad))).reshape(
        num_op, NW, k_chunks, CHUNK)
    val = jnp.pad(adj_values, ((0, 0), (0, pad))).reshape(
        num_op, NW, k_chunks, CHUNK)
    wsp = jnp.tile((ws / jnp.float32(num_op))[:, None], (1, LANES))

    partials = _sc_spmm(num_op, n, d, k_chunks)(x, src, dst, val, wsp)

    blk = 1000
    return pl.pallas_call(
        _combine,
        grid=(n // blk,),
        in_specs=[pl.BlockSpec((NUM_CORES, blk, d), lambda i: (0, i, 0))],
        out_specs=pl.BlockSpec((blk, d), lambda i: (i, 0)),
        out_shape=jax.ShapeDtypeStruct((n, d), jnp.float32),
    )(partials)


# speedup vs baseline: 1.6288x; 1.2347x over previous
import functools

import jax
import jax.numpy as jnp
from jax import lax
from jax.experimental import pallas as pl
from jax.experimental.pallas import tpu as pltpu
from jax.experimental.pallas import tpu_sc as plsc

LANES = 16
NUM_CORES = 2
NUM_SUBCORES = 16
CHUNK = 128


def _sc_spmm(num_op, n, d, k_chunks):
    dh = d // NUM_CORES
    qs = dh // LANES
    zcopy = CHUNK
    n_zcopy = -(-n // (NUM_SUBCORES * zcopy))
    rows_per_tile = n_zcopy * zcopy
    n_acc = NUM_SUBCORES * rows_per_tile

    mesh = plsc.VectorSubcoreMesh(core_axis_name="c", subcore_axis_name="s")

    @functools.partial(
        pl.kernel,
        mesh=mesh,
        compiler_params=pltpu.CompilerParams(
            needs_layout_passes=False, use_tc_tiling_on_sc=False),
        out_type=jax.ShapeDtypeStruct((NUM_CORES, n_acc, dh), jnp.float32),
        scratch_types=[
            pltpu.VMEM((num_op, LANES), jnp.float32),
            pltpu.VMEM((k_chunks, CHUNK), jnp.int32),
            pltpu.VMEM((k_chunks, CHUNK), jnp.int32),
            pltpu.VMEM((k_chunks, CHUNK), jnp.float32),
            pltpu.VMEM((CHUNK, dh), jnp.float32),
            pltpu.VMEM((CHUNK, dh), jnp.float32),
            pltpu.VMEM((CHUNK, dh), jnp.float32),
            pltpu.VMEM_SHARED((n_acc, dh), jnp.float32),
            pltpu.SemaphoreType.DMA,
            pltpu.SemaphoreType.DMA,
            pltpu.SemaphoreType.DMA,
            pltpu.SemaphoreType.DMA,
            pltpu.SemaphoreType.DMA,
            pltpu.SemaphoreType.DMA,
        ],
    )
    def k(x_hbm, src_hbm, dst_hbm, val_hbm, ws_hbm, out_hbm,
          ws_v, src_v, dst_v, val_v, rows_0, rows_1, rows_2, acc,
          sem_0, sem_1, sem_2, ssem_0, ssem_1, ssem_2):
        bufs = (rows_0, rows_1, rows_2)
        sems = (sem_0, sem_1, sem_2)
        ssems = (ssem_0, ssem_1, ssem_2)
        c = lax.axis_index("c")
        s = lax.axis_index("s")

        def _zrow(r, carry):
            for q in range(qs):
                rows_0[r, pl.ds(q * LANES, LANES)] = jnp.zeros(
                    (LANES,), jnp.float32)
            return carry
        lax.fori_loop(0, CHUNK, _zrow, 0)

        base = s * rows_per_tile
        for z in range(n_zcopy):
            pltpu.sync_copy(rows_0.at[pl.ds(0, zcopy)],
                            acc.at[pl.ds(base + z * zcopy, zcopy)])
        plsc.subcore_barrier()

        pltpu.sync_copy(ws_hbm, ws_v)

        for i in range(num_op):
            pltpu.sync_copy(src_hbm.at[c, i, s], src_v)
            pltpu.sync_copy(dst_hbm.at[i, s], dst_v)
            pltpu.sync_copy(val_hbm.at[i, s], val_v)
            wvec = ws_v[i]

            def _vscale(jj, carry):
                for q in range(CHUNK // LANES):
                    sl = pl.ds(q * LANES, LANES)
                    val_v[jj, sl] = val_v[jj, sl] * wvec
                return carry
            lax.fori_loop(0, k_chunks, _vscale, 0)

            pltpu.async_copy(x_hbm.at[src_v.at[0]], bufs[0], sems[0])
            pltpu.async_copy(x_hbm.at[src_v.at[1]], bufs[1], sems[1])

            def _trip(t, carry):
                for b in range(3):
                    j = 3 * t + b
                    nb = (b + 2) % 3
                    rows_b = bufs[b]
                    pltpu.make_async_copy(
                        x_hbm.at[src_v.at[j]], rows_b, sems[b]).wait()

                    @pl.when(j >= 1)
                    def _():
                        pltpu.make_async_copy(
                            bufs[nb], acc.at[dst_v.at[j - 1]],
                            ssems[nb]).wait()

                    @pl.when(j + 2 < k_chunks)
                    def _():
                        pltpu.async_copy(
                            x_hbm.at[src_v.at[j + 2]], bufs[nb],
                            sems[nb])

                    @plsc.parallel_loop(0, CHUNK // LANES)
                    def _grp(g):
                        vv = val_v[j, pl.ds(g * LANES, LANES)]
                        rbase = g * LANES
                        for l in range(LANES):
                            sval = vv[l]
                            r = rbase + l
                            loaded = [rows_b[r, pl.ds(q * LANES, LANES)]
                                      for q in range(qs)]
                            prods = [v * sval for v in loaded]
                            for q in range(qs):
                                rows_b[r, pl.ds(q * LANES, LANES)] = prods[q]

                    pltpu.async_copy(
                        rows_b, acc.at[dst_v.at[j]], ssems[b], add=True)
                return carry
            lax.fori_loop(0, k_chunks // 3, _trip, 0)

            pltpu.make_async_copy(
                bufs[2], acc.at[dst_v.at[k_chunks - 1]], ssems[2]).wait()

        plsc.subcore_barrier()
        for z in range(n_zcopy):
            sl = pl.ds(base + z * zcopy, zcopy)
            pltpu.sync_copy(acc.at[sl], out_hbm.at[c].at[sl])

    return k


def kernel(x, adj_indices, adj_values, ws):
    n, d = x.shape
    num_op, _, e = adj_indices.shape
    dh = d // NUM_CORES
    k_chunks = -(-e // (NUM_SUBCORES * CHUNK))
    k_chunks = 3 * (-(-k_chunks // 3))
    e_pad = NUM_SUBCORES * k_chunks * CHUNK
    pad = e_pad - e

    xh = x.reshape(n, NUM_CORES, dh).transpose(1, 0, 2).reshape(
        NUM_CORES * n, dh)
    src = jnp.pad(adj_indices[:, 1, :], ((0, 0), (0, pad))).reshape(
        num_op, NUM_SUBCORES, k_chunks, CHUNK)
    src = jnp.stack([src + cc * n for cc in range(NUM_CORES)])
    dst = jnp.pad(adj_indices[:, 0, :], ((0, 0), (0, pad))).reshape(
        num_op, NUM_SUBCORES, k_chunks, CHUNK)
    val = jnp.pad(adj_values, ((0, 0), (0, pad))).reshape(
        num_op, NUM_SUBCORES, k_chunks, CHUNK)
    wsp = jnp.tile((ws / jnp.float32(num_op))[:, None], (1, LANES))

    parts = _sc_spmm(num_op, n, d, k_chunks)(xh, src, dst, val, wsp)
    return parts.transpose(1, 0, 2).reshape(-1, d)[:n]


# speedup vs baseline: 1.6988x; 1.0430x over previous
import functools

import jax
import jax.numpy as jnp
from jax import lax
from jax.experimental import pallas as pl
from jax.experimental.pallas import tpu as pltpu
from jax.experimental.pallas import tpu_sc as plsc

LANES = 16
NUM_CORES = 2
NUM_SUBCORES = 16
CHUNK = 128


def _sc_spmm(num_op, n, d, k_chunks):
    dh = d // NUM_CORES
    qs = dh // LANES
    zcopy = CHUNK
    n_zcopy = -(-n // (NUM_SUBCORES * zcopy))
    rows_per_tile = n_zcopy * zcopy
    n_acc = NUM_SUBCORES * rows_per_tile

    mesh = plsc.VectorSubcoreMesh(core_axis_name="c", subcore_axis_name="s")

    @functools.partial(
        pl.kernel,
        mesh=mesh,
        compiler_params=pltpu.CompilerParams(
            needs_layout_passes=False, use_tc_tiling_on_sc=False),
        out_type=jax.ShapeDtypeStruct((NUM_CORES, n_acc, dh), jnp.float32),
        scratch_types=[
            pltpu.VMEM((num_op, LANES), jnp.float32),
            pltpu.VMEM((k_chunks, CHUNK), jnp.int32),
            pltpu.VMEM((k_chunks, CHUNK), jnp.int32),
            pltpu.VMEM((k_chunks, CHUNK), jnp.float32),
            pltpu.VMEM((CHUNK, dh), jnp.float32),
            pltpu.VMEM((CHUNK, dh), jnp.float32),
            pltpu.VMEM_SHARED((n_acc, dh), jnp.float32),
            pltpu.SemaphoreType.DMA,
            pltpu.SemaphoreType.DMA,
            pltpu.SemaphoreType.DMA,
            pltpu.SemaphoreType.DMA,
        ],
    )
    def k(x_hbm, src_hbm, dst_hbm, val_hbm, ws_hbm, out_hbm,
          ws_v, src_v, dst_v, val_v, rows_0, rows_1, acc,
          sem_0, sem_1, ssem_0, ssem_1):
        bufs = (rows_0, rows_1)
        sems = (sem_0, sem_1)
        ssems = (ssem_0, ssem_1)
        c = lax.axis_index("c")
        s = lax.axis_index("s")

        def _zrow(r, carry):
            for q in range(qs):
                rows_0[r, pl.ds(q * LANES, LANES)] = jnp.zeros(
                    (LANES,), jnp.float32)
            return carry
        lax.fori_loop(0, CHUNK, _zrow, 0)

        base = s * rows_per_tile
        for z in range(n_zcopy):
            pltpu.sync_copy(rows_0.at[pl.ds(0, zcopy)],
                            acc.at[pl.ds(base + z * zcopy, zcopy)])
        plsc.subcore_barrier()

        pltpu.sync_copy(ws_hbm, ws_v)

        for i in range(num_op):
            pltpu.sync_copy(src_hbm.at[c, i, s], src_v)
            pltpu.sync_copy(dst_hbm.at[i, s], dst_v)
            pltpu.sync_copy(val_hbm.at[i, s], val_v)
            wvec = ws_v[i]

            def _vscale(jj, carry):
                for q in range(CHUNK // LANES):
                    sl = pl.ds(q * LANES, LANES)
                    val_v[jj, sl] = val_v[jj, sl] * wvec
                return carry
            lax.fori_loop(0, k_chunks, _vscale, 0)

            pltpu.async_copy(x_hbm.at[src_v.at[0]], bufs[0], sems[0])

            def _pair(t, carry):
                for b in range(2):
                    j = 2 * t + b
                    rows_b = bufs[b]
                    pltpu.make_async_copy(
                        x_hbm.at[src_v.at[j]], rows_b, sems[b]).wait()

                    @pl.when(j >= 1)
                    def _():
                        pltpu.make_async_copy(
                            bufs[1 - b], acc.at[dst_v.at[j - 1]],
                            ssems[1 - b]).wait()

                    @pl.when(j + 1 < k_chunks)
                    def _():
                        pltpu.async_copy(
                            x_hbm.at[src_v.at[j + 1]], bufs[1 - b],
                            sems[1 - b])

                    @plsc.parallel_loop(0, CHUNK // LANES)
                    def _grp(g):
                        vv = val_v[j, pl.ds(g * LANES, LANES)]
                        rbase = g * LANES
                        for l in range(LANES):
                            sval = vv[l]
                            r = rbase + l
                            loaded = [rows_b[r, pl.ds(q * LANES, LANES)]
                                      for q in range(qs)]
                            prods = [v * sval for v in loaded]
                            for q in range(qs):
                                rows_b[r, pl.ds(q * LANES, LANES)] = prods[q]

                    pltpu.async_copy(
                        rows_b, acc.at[dst_v.at[j]], ssems[b], add=True)
                return carry
            lax.fori_loop(0, k_chunks // 2, _pair, 0)

            pltpu.make_async_copy(
                bufs[1], acc.at[dst_v.at[k_chunks - 1]], ssems[1]).wait()

        plsc.subcore_barrier()
        for z in range(n_zcopy):
            sl = pl.ds(base + z * zcopy, zcopy)
            pltpu.sync_copy(acc.at[sl], out_hbm.at[c].at[sl])

    return k


def kernel(x, adj_indices, adj_values, ws):
    n, d = x.shape
    num_op, _, e = adj_indices.shape
    dh = d // NUM_CORES
    k_chunks = -(-e // (NUM_SUBCORES * CHUNK))
    k_chunks += k_chunks % 2
    e_pad = NUM_SUBCORES * k_chunks * CHUNK
    pad = e_pad - e

    xh = x.reshape(n, NUM_CORES, dh).transpose(1, 0, 2).reshape(
        NUM_CORES * n, dh)
    src = jnp.pad(adj_indices[:, 1, :], ((0, 0), (0, pad))).reshape(
        num_op, NUM_SUBCORES, k_chunks, CHUNK)
    src = jnp.stack([src + cc * n for cc in range(NUM_CORES)])
    dst = jnp.pad(adj_indices[:, 0, :], ((0, 0), (0, pad))).reshape(
        num_op, NUM_SUBCORES, k_chunks, CHUNK)
    val = jnp.pad(adj_values, ((0, 0), (0, pad))).reshape(
        num_op, NUM_SUBCORES, k_chunks, CHUNK)
    wsp = jnp.tile((ws / jnp.float32(num_op))[:, None], (1, LANES))

    parts = _sc_spmm(num_op, n, d, k_chunks)(xh, src, dst, val, wsp)
    return parts.transpose(1, 0, 2).reshape(-1, d)[:n]
